# baseline (speedup 1.0000x reference)
import functools

import jax
import jax.numpy as jnp
from jax.experimental import pallas as pl
from jax.experimental.pallas import tpu as pltpu

D_MODEL = 36
HEAD_DIM = 2
V_DIM = 1
N_TOTAL = 32768
BLOCK_N = 4096


def _body(qe_ref, wq_ref, wk_ref, wv_ref, m_ref,
          val_ref, score_ref, idx_ref,
          best_score, best_idx, best_row):
    k = pl.program_id(0)
    nblk = pl.num_programs(0)

    qe = qe_ref[...]
    wq = wq_ref[...]
    wk = wk_ref[...]
    q = jnp.sum(wq * qe, axis=1, keepdims=True)
    c = jnp.sum(wk * q, axis=0, keepdims=True)

    m = m_ref[...]
    scores = jnp.sum(m * c, axis=1, keepdims=True)

    local_max = jnp.max(scores)
    rows = jax.lax.broadcasted_iota(jnp.int32, scores.shape, 0)
    local_idx = jnp.min(jnp.where(scores == local_max, rows, N_TOTAL))

    cur = jnp.where(k == 0, -jnp.inf, best_score[0, 0])

    @pl.when(local_max > cur)
    def _update():
        best_score[0, 0] = local_max
        best_idx[0, 0] = k * BLOCK_N + local_idx
        rows2 = jax.lax.broadcasted_iota(jnp.int32, m.shape, 0)
        best_row[...] = jnp.sum(
            jnp.where(rows2 == local_idx, m, 0.0), axis=0, keepdims=True)

    @pl.when(k == nblk - 1)
    def _finish():
        wv = wv_ref[...]
        row = best_row[...]
        val_ref[...] = jnp.sum(wv * row, axis=1, keepdims=True)
        score_ref[...] = jnp.full((1, 1), best_score[0, 0], jnp.float32)
        idx_ref[...] = jnp.full((1, 1), best_idx[0, 0], jnp.int32)


@jax.jit
def kernel(query_emb, memory_embs, W_Q, W_K, W_V):
    n = memory_embs.shape[0]
    nblk = n // BLOCK_N
    small = lambda shape: pl.BlockSpec(shape, lambda k: (0, 0))
    value, score, best = pl.pallas_call(
        _body,
        grid=(nblk,),
        in_specs=[
            small((1, D_MODEL)),
            small((HEAD_DIM, D_MODEL)),
            small((HEAD_DIM, D_MODEL)),
            small((V_DIM, D_MODEL)),
            pl.BlockSpec((BLOCK_N, D_MODEL), lambda k: (k, 0)),
        ],
        out_specs=[small((V_DIM, 1)), small((1, 1)), small((1, 1))],
        out_shape=[
            jax.ShapeDtypeStruct((V_DIM, 1), jnp.float32),
            jax.ShapeDtypeStruct((1, 1), jnp.float32),
            jax.ShapeDtypeStruct((1, 1), jnp.int32),
        ],
        scratch_shapes=[
            pltpu.SMEM((1, 1), jnp.float32),
            pltpu.SMEM((1, 1), jnp.int32),
            pltpu.VMEM((1, D_MODEL), jnp.float32),
        ],
    )(query_emb.reshape(1, D_MODEL), W_Q, W_K, W_V, memory_embs)
    return (value.reshape(V_DIM), score[0, 0], best[0, 0])


# speedup vs baseline: 1.0022x; 1.0022x over previous
import functools

import jax
import jax.numpy as jnp
from jax.experimental import pallas as pl
from jax.experimental.pallas import tpu as pltpu

D_MODEL = 36
HEAD_DIM = 2
V_DIM = 1
N_TOTAL = 32768
BLOCK_N = 4096


def _body(qe_ref, wq_ref, wk_ref, wv_ref, m_ref,
          val_ref, score_ref, idx_ref,
          best_score, best_idx, best_row):
    k = pl.program_id(0)
    nblk = pl.num_programs(0)

    qe = qe_ref[...]
    wq = wq_ref[...]
    wk = wk_ref[...]
    q = jnp.sum(wq * qe, axis=1, keepdims=True)
    c = jnp.sum(wk * q, axis=0, keepdims=True)

    m = m_ref[...]
    scores = jax.lax.dot_general(
        m, c, (((1,), (1,)), ((), ())),
        preferred_element_type=jnp.float32)

    local_max = jnp.max(scores)
    rows = jax.lax.broadcasted_iota(jnp.int32, scores.shape, 0)
    local_idx = jnp.min(jnp.where(scores == local_max, rows, N_TOTAL))

    cur = jnp.where(k == 0, -jnp.inf, best_score[0, 0])

    @pl.when(local_max > cur)
    def _update():
        best_score[0, 0] = local_max
        best_idx[0, 0] = k * BLOCK_N + local_idx
        rows2 = jax.lax.broadcasted_iota(jnp.int32, m.shape, 0)
        best_row[...] = jnp.sum(
            jnp.where(rows2 == local_idx, m, 0.0), axis=0, keepdims=True)

    @pl.when(k == nblk - 1)
    def _finish():
        wv = wv_ref[...]
        row = best_row[...]
        val_ref[...] = jnp.sum(wv * row, axis=1, keepdims=True)
        score_ref[...] = jnp.full((1, 1), best_score[0, 0], jnp.float32)
        idx_ref[...] = jnp.full((1, 1), best_idx[0, 0], jnp.int32)


@jax.jit
def kernel(query_emb, memory_embs, W_Q, W_K, W_V):
    n = memory_embs.shape[0]
    nblk = n // BLOCK_N
    small = lambda shape: pl.BlockSpec(shape, lambda k: (0, 0))
    value, score, best = pl.pallas_call(
        _body,
        grid=(nblk,),
        in_specs=[
            small((1, D_MODEL)),
            small((HEAD_DIM, D_MODEL)),
            small((HEAD_DIM, D_MODEL)),
            small((V_DIM, D_MODEL)),
            pl.BlockSpec((BLOCK_N, D_MODEL), lambda k: (k, 0)),
        ],
        out_specs=[small((V_DIM, 1)), small((1, 1)), small((1, 1))],
        out_shape=[
            jax.ShapeDtypeStruct((V_DIM, 1), jnp.float32),
            jax.ShapeDtypeStruct((1, 1), jnp.float32),
            jax.ShapeDtypeStruct((1, 1), jnp.int32),
        ],
        scratch_shapes=[
            pltpu.SMEM((1, 1), jnp.float32),
            pltpu.SMEM((1, 1), jnp.int32),
            pltpu.VMEM((1, D_MODEL), jnp.float32),
        ],
    )(query_emb.reshape(1, D_MODEL), W_Q, W_K, W_V, memory_embs)
    return (value.reshape(V_DIM), score[0, 0], best[0, 0])
